# Initial kernel scaffold; baseline (speedup 1.0000x reference)
import jax
import jax.numpy as jnp
from jax.experimental import pallas as pl


def kernel(x, Wg1, bg1, Wg2, bg2, W1, b1, W2, b2):
    raise NotImplementedError("write your pallas kernel here")


# speedup vs baseline: 12.9398x; 12.9398x over previous
import functools

import jax
import jax.numpy as jnp
from jax import lax
from jax.experimental import pallas as pl
from jax.experimental.pallas import tpu as pltpu
from jax.experimental.pallas import tpu_sc as plsc

_INTERPRET = False

D_MODEL = 768
D_FF = 4 * D_MODEL
E = 64
S = 2048
TBLK = 256
T = 48
MAXB = S // T + E
P = MAXB * T


_NB = S // TBLK


def _route_body(x_ref, wg1_ref, bg1_ref, wg2_ref, bg2_ref,
                p_ref, be_ref, nact_ref, lb_ref,
                ids_s, usage_acc, cnt_row_acc, cnt_col_acc, carry):
    b = pl.program_id(0)

    @pl.when(b == 0)
    def _init():
        usage_acc[...] = jnp.zeros_like(usage_acc)
        cnt_row_acc[...] = jnp.zeros_like(cnt_row_acc)
        cnt_col_acc[...] = jnp.zeros_like(cnt_col_acc)
        carry[...] = jnp.zeros_like(carry)

    @pl.when(b < _NB)
    def _gate_phase():
        x = x_ref[...]
        h = jnp.maximum(
            jnp.dot(x, wg1_ref[...], preferred_element_type=jnp.float32)
            + bg1_ref[...], 0.0)
        logits = (jnp.dot(h, wg2_ref[...],
                          preferred_element_type=jnp.float32)
                  + bg2_ref[...])
        m = jnp.max(logits, axis=-1, keepdims=True)
        ex = jnp.exp(logits - m)
        probs = ex / jnp.sum(ex, axis=-1, keepdims=True)
        ids = jnp.argmax(logits, axis=-1).astype(jnp.int32)
        ids_s[pl.ds(b, 1), :] = ids.reshape(1, TBLK)

        oh = (lax.broadcasted_iota(jnp.int32, (TBLK, E), 1)
              == ids[:, None]).astype(jnp.float32)
        oh_col = (lax.broadcasted_iota(jnp.int32, (E, TBLK), 0)
                  == ids.reshape(1, TBLK)).astype(jnp.float32)
        usage_acc[...] += jnp.sum(probs, axis=0, keepdims=True)
        cnt_row_acc[...] += jnp.sum(oh, axis=0, keepdims=True)
        cnt_col_acc[...] += jnp.dot(oh_col,
                                    jnp.ones((TBLK, 1), jnp.float32),
                                    preferred_element_type=jnp.float32)

        @pl.when(b == _NB - 1)
        def _fin():
            usage = usage_acc[...] / S
            lb_ref[...] = 0.01 * jnp.mean((usage - 1.0 / E) ** 2,
                                          keepdims=True).reshape(1, 1)

    @pl.when(b >= _NB)
    def _meta_phase():
        j = b - _NB
        counts_row = cnt_row_acc[...]
        blocks_row = jnp.floor((counts_row + (T - 1)) * (1.0 / T))
        r = lax.broadcasted_iota(jnp.int32, (E, E), 0)
        c = lax.broadcasted_iota(jnp.int32, (E, E), 1)
        strict_lo = (r < c).astype(jnp.float32)
        excl_row = jnp.dot(blocks_row, strict_lo,
                           preferred_element_type=jnp.float32)
        base_row = excl_row * T

        @pl.when(b == _NB)
        def _once():
            counts_col = cnt_col_acc[...]
            blocks_col = jnp.floor((counts_col + (T - 1)) * (1.0 / T))
            incl_lo = (r >= c).astype(jnp.float32)
            incl_col = jnp.dot(incl_lo, blocks_col,
                               preferred_element_type=jnp.float32)
            bidx = lax.broadcasted_iota(jnp.int32,
                                        (E, MAXB), 1).astype(jnp.float32)
            g = (bidx >= incl_col).astype(jnp.float32)
            be = jnp.sum(g, axis=0, keepdims=True)
            be_ref[...] = jnp.minimum(be, E - 1).astype(jnp.int32)
            nact_ref[...] = jnp.sum(blocks_row,
                                    keepdims=True).astype(jnp.int32)

        ids = ids_s[pl.ds(j, 1), :].reshape(TBLK)
        oh = (lax.broadcasted_iota(jnp.int32, (TBLK, E), 1)
              == ids[:, None]).astype(jnp.float32)
        ti = lax.broadcasted_iota(jnp.int32, (TBLK, TBLK), 0)
        tj = lax.broadcasted_iota(jnp.int32, (TBLK, TBLK), 1)
        tri = (tj < ti).astype(jnp.float32)
        rank_within = jnp.dot(tri, oh,
                              preferred_element_type=jnp.float32)
        p_val = jnp.sum(oh * (rank_within + base_row + carry[...]),
                        axis=1)
        p_ref[...] = p_val.astype(jnp.int32).reshape(1, 1, TBLK)
        carry[...] += jnp.sum(oh, axis=0, keepdims=True)


def _route_call(xf, Wg1, bg1, Wg2, bg2):
    return pl.pallas_call(
        _route_body,
        grid=(2 * _NB,),
        in_specs=[
            pl.BlockSpec((TBLK, D_MODEL), lambda b: (jnp.minimum(b, _NB - 1), 0)),
            pl.BlockSpec((D_MODEL, 2 * D_MODEL), lambda b: (0, 0)),
            pl.BlockSpec((1, 2 * D_MODEL), lambda b: (0, 0)),
            pl.BlockSpec((2 * D_MODEL, E), lambda b: (0, 0)),
            pl.BlockSpec((1, E), lambda b: (0, 0)),
        ],
        out_specs=[
            pl.BlockSpec((1, 1, TBLK),
                         lambda b: (jnp.maximum(b - _NB, 0), 0, 0)),
            pl.BlockSpec((1, MAXB), lambda b: (0, 0)),
            pl.BlockSpec((1, 1), lambda b: (0, 0)),
            pl.BlockSpec((1, 1), lambda b: (0, 0)),
        ],
        out_shape=[
            jax.ShapeDtypeStruct((_NB, 1, TBLK), jnp.int32),
            jax.ShapeDtypeStruct((1, MAXB), jnp.int32),
            jax.ShapeDtypeStruct((1, 1), jnp.int32),
            jax.ShapeDtypeStruct((1, 1), jnp.float32),
        ],
        scratch_shapes=[
            pltpu.VMEM((_NB, TBLK), jnp.int32),
            pltpu.VMEM((1, E), jnp.float32),
            pltpu.VMEM((1, E), jnp.float32),
            pltpu.VMEM((E, 1), jnp.float32),
            pltpu.VMEM((1, E), jnp.float32),
        ],
        interpret=_INTERPRET,
    )(xf, Wg1, bg1, Wg2, bg2)


_NW = 32
_ROWS_W = S // _NW


def _dispatch_call(xf, p):
    mesh = plsc.VectorSubcoreMesh(core_axis_name="c", subcore_axis_name="s")

    @functools.partial(
        pl.kernel, mesh=mesh,
        out_type=jax.ShapeDtypeStruct((P, D_MODEL), jnp.float32),
        scratch_types=[
            pltpu.VMEM((_ROWS_W,), jnp.int32),
            pltpu.VMEM((_ROWS_W, D_MODEL), jnp.float32),
        ],
    )
    def disp(xf_hbm, p_hbm, out_hbm, idx_v, rows_v):
        wid = lax.axis_index("s") * 2 + lax.axis_index("c")
        base = wid * _ROWS_W
        pltpu.sync_copy(p_hbm.at[pl.ds(base, _ROWS_W)], idx_v)
        pltpu.sync_copy(xf_hbm.at[pl.ds(base, _ROWS_W)], rows_v)
        pltpu.sync_copy(rows_v, out_hbm.at[idx_v])

    return disp(xf, p)


def _combine_call(y_pad, p):
    mesh = plsc.VectorSubcoreMesh(core_axis_name="c", subcore_axis_name="s")

    @functools.partial(
        pl.kernel, mesh=mesh,
        out_type=jax.ShapeDtypeStruct((S, D_MODEL), jnp.float32),
        scratch_types=[
            pltpu.VMEM((_ROWS_W,), jnp.int32),
            pltpu.VMEM((_ROWS_W, D_MODEL), jnp.float32),
            pltpu.SemaphoreType.DMA,
        ],
    )
    def comb(ypad_hbm, p_hbm, out_hbm, idx_v, rows_v, sem):
        wid = lax.axis_index("s") * 2 + lax.axis_index("c")
        base = wid * _ROWS_W
        pltpu.sync_copy(p_hbm.at[pl.ds(base, _ROWS_W)], idx_v)
        pltpu.async_copy(ypad_hbm.at[idx_v], rows_v, sem).wait()
        pltpu.sync_copy(rows_v, out_hbm.at[pl.ds(base, _ROWS_W)])

    return comb(y_pad, p)


def _ffn_body(be_ref, nact_ref, x_ref, w1_ref, b1_ref, w2_ref, b2_ref,
              out_ref):
    b = pl.program_id(0)

    @pl.when(b < nact_ref[0])
    def _active():
        e = be_ref[jnp.minimum(b, nact_ref[0] - 1)]
        x = x_ref[...]
        h = (jnp.dot(x, w1_ref[0], preferred_element_type=jnp.float32,
                     precision=lax.Precision.DEFAULT)
             + b1_ref[pl.ds(e, 1), :])
        h = 0.5 * h * (1.0 + lax.erf(h * 0.7071067811865476))
        out_ref[...] = (jnp.dot(h, w2_ref[0],
                                preferred_element_type=jnp.float32,
                                precision=lax.Precision.DEFAULT)
                        + b2_ref[pl.ds(e, 1), :])


def _ffn_call(block_expert, nact, x_pad, W1, b1, W2, b2):
    def _rowblk(b, be, na):
        return (jnp.minimum(b, na[0] - 1), 0)

    def _expert3(b, be, na):
        return (be[jnp.minimum(b, na[0] - 1)], 0, 0)

    grid_spec = pltpu.PrefetchScalarGridSpec(
        num_scalar_prefetch=2,
        grid=(MAXB,),
        in_specs=[
            pl.BlockSpec((T, D_MODEL), _rowblk),
            pl.BlockSpec((1, D_MODEL, D_FF), _expert3),
            pl.BlockSpec((E, D_FF), lambda b, be, na: (0, 0)),
            pl.BlockSpec((1, D_FF, D_MODEL), _expert3),
            pl.BlockSpec((E, D_MODEL), lambda b, be, na: (0, 0)),
        ],
        out_specs=pl.BlockSpec((T, D_MODEL), _rowblk),
    )
    return pl.pallas_call(
        _ffn_body,
        grid_spec=grid_spec,
        out_shape=jax.ShapeDtypeStruct((P, D_MODEL), jnp.float32),
        interpret=_INTERPRET,
    )(block_expert, nact, x_pad, W1, b1, W2, b2)


def kernel(x, Wg1, bg1, Wg2, bg2, W1, b1, W2, b2):
    B, S_, D = x.shape
    xf = x.reshape(S_, D)

    p3, be, nact, lb = _route_call(
        xf, Wg1, bg1.reshape(1, -1), Wg2, bg2.reshape(1, -1))
    p = p3.reshape(S_)
    block_expert = be.reshape(MAXB)

    x_pad = _dispatch_call(xf, p)
    y_pad = _ffn_call(block_expert, nact.reshape(1), x_pad, W1, b1, W2, b2)
    y = _combine_call(y_pad, p)

    return y.reshape(B, S_, D), lb.reshape(())


# speedup vs baseline: 13.4587x; 1.0401x over previous
import functools

import jax
import jax.numpy as jnp
from jax import lax
from jax.experimental import pallas as pl
from jax.experimental.pallas import tpu as pltpu
from jax.experimental.pallas import tpu_sc as plsc

D_MODEL = 768
D_FF = 4 * D_MODEL
E = 64
S = 2048
TBLK = 256
T = 64
MAXB = S // T + E
P = MAXB * T


_NB = S // TBLK


def _route_body(x_ref, wg1_ref, bg1_ref, wg2_ref, bg2_ref,
                p_ref, be_ref, nact_ref, lb_ref,
                ids_s, usage_acc, cnt_row_acc, cnt_col_acc, carry):
    b = pl.program_id(0)

    @pl.when(b == 0)
    def _init():
        usage_acc[...] = jnp.zeros_like(usage_acc)
        cnt_row_acc[...] = jnp.zeros_like(cnt_row_acc)
        cnt_col_acc[...] = jnp.zeros_like(cnt_col_acc)
        carry[...] = jnp.zeros_like(carry)

    @pl.when(b < _NB)
    def _gate_phase():
        x = x_ref[...]
        h = jnp.maximum(
            jnp.dot(x, wg1_ref[...], preferred_element_type=jnp.float32)
            + bg1_ref[...], 0.0)
        logits = (jnp.dot(h, wg2_ref[...],
                          preferred_element_type=jnp.float32)
                  + bg2_ref[...])
        m = jnp.max(logits, axis=-1, keepdims=True)
        ex = jnp.exp(logits - m)
        probs = ex / jnp.sum(ex, axis=-1, keepdims=True)
        ids = jnp.argmax(logits, axis=-1).astype(jnp.int32)
        ids_s[pl.ds(b, 1), :] = ids.reshape(1, TBLK)

        oh = (lax.broadcasted_iota(jnp.int32, (TBLK, E), 1)
              == ids[:, None]).astype(jnp.float32)
        oh_col = (lax.broadcasted_iota(jnp.int32, (E, TBLK), 0)
                  == ids.reshape(1, TBLK)).astype(jnp.float32)
        usage_acc[...] += jnp.sum(probs, axis=0, keepdims=True)
        cnt_row_acc[...] += jnp.sum(oh, axis=0, keepdims=True)
        cnt_col_acc[...] += jnp.dot(oh_col,
                                    jnp.ones((TBLK, 1), jnp.float32),
                                    preferred_element_type=jnp.float32)

        @pl.when(b == _NB - 1)
        def _fin():
            usage = usage_acc[...] / S
            lb_ref[...] = 0.01 * jnp.mean((usage - 1.0 / E) ** 2,
                                          keepdims=True).reshape(1, 1)

    @pl.when(b >= _NB)
    def _meta_phase():
        j = b - _NB
        counts_row = cnt_row_acc[...]
        blocks_row = jnp.floor((counts_row + (T - 1)) * (1.0 / T))
        r = lax.broadcasted_iota(jnp.int32, (E, E), 0)
        c = lax.broadcasted_iota(jnp.int32, (E, E), 1)
        strict_lo = (r < c).astype(jnp.float32)
        excl_row = jnp.dot(blocks_row, strict_lo,
                           preferred_element_type=jnp.float32)
        base_row = excl_row * T

        @pl.when(b == _NB)
        def _once():
            counts_col = cnt_col_acc[...]
            blocks_col = jnp.floor((counts_col + (T - 1)) * (1.0 / T))
            incl_lo = (r >= c).astype(jnp.float32)
            incl_col = jnp.dot(incl_lo, blocks_col,
                               preferred_element_type=jnp.float32)
            bidx = lax.broadcasted_iota(jnp.int32,
                                        (E, MAXB), 1).astype(jnp.float32)
            g = (bidx >= incl_col).astype(jnp.float32)
            be = jnp.sum(g, axis=0, keepdims=True)
            be_ref[...] = jnp.minimum(be, E - 1).astype(jnp.int32)
            nact_ref[...] = jnp.sum(blocks_row,
                                    keepdims=True).astype(jnp.int32)

        ids = ids_s[pl.ds(j, 1), :].reshape(TBLK)
        oh = (lax.broadcasted_iota(jnp.int32, (TBLK, E), 1)
              == ids[:, None]).astype(jnp.float32)
        ti = lax.broadcasted_iota(jnp.int32, (TBLK, TBLK), 0)
        tj = lax.broadcasted_iota(jnp.int32, (TBLK, TBLK), 1)
        tri = (tj < ti).astype(jnp.float32)
        rank_within = jnp.dot(tri, oh,
                              preferred_element_type=jnp.float32)
        p_val = jnp.sum(oh * (rank_within + base_row + carry[...]),
                        axis=1)
        p_ref[...] = p_val.astype(jnp.int32).reshape(1, 1, TBLK)
        carry[...] += jnp.sum(oh, axis=0, keepdims=True)


def _route_call(xf, Wg1, bg1, Wg2, bg2):
    return pl.pallas_call(
        _route_body,
        grid=(2 * _NB,),
        in_specs=[
            pl.BlockSpec((TBLK, D_MODEL), lambda b: (jnp.minimum(b, _NB - 1), 0)),
            pl.BlockSpec((D_MODEL, 2 * D_MODEL), lambda b: (0, 0)),
            pl.BlockSpec((1, 2 * D_MODEL), lambda b: (0, 0)),
            pl.BlockSpec((2 * D_MODEL, E), lambda b: (0, 0)),
            pl.BlockSpec((1, E), lambda b: (0, 0)),
        ],
        out_specs=[
            pl.BlockSpec((1, 1, TBLK),
                         lambda b: (jnp.maximum(b - _NB, 0), 0, 0)),
            pl.BlockSpec((1, MAXB), lambda b: (0, 0)),
            pl.BlockSpec((1, 1), lambda b: (0, 0)),
            pl.BlockSpec((1, 1), lambda b: (0, 0)),
        ],
        out_shape=[
            jax.ShapeDtypeStruct((_NB, 1, TBLK), jnp.int32),
            jax.ShapeDtypeStruct((1, MAXB), jnp.int32),
            jax.ShapeDtypeStruct((1, 1), jnp.int32),
            jax.ShapeDtypeStruct((1, 1), jnp.float32),
        ],
        scratch_shapes=[
            pltpu.VMEM((_NB, TBLK), jnp.int32),
            pltpu.VMEM((1, E), jnp.float32),
            pltpu.VMEM((1, E), jnp.float32),
            pltpu.VMEM((E, 1), jnp.float32),
            pltpu.VMEM((1, E), jnp.float32),
        ],
    )(xf, Wg1, bg1, Wg2, bg2)


_NW = 32
_ROWS_W = S // _NW


def _dispatch_call(xf, p):
    mesh = plsc.VectorSubcoreMesh(core_axis_name="c", subcore_axis_name="s")

    @functools.partial(
        pl.kernel, mesh=mesh,
        out_type=jax.ShapeDtypeStruct((P, D_MODEL), jnp.float32),
        scratch_types=[
            pltpu.VMEM((_ROWS_W,), jnp.int32),
            pltpu.VMEM((_ROWS_W, D_MODEL), jnp.float32),
        ],
    )
    def disp(xf_hbm, p_hbm, out_hbm, idx_v, rows_v):
        wid = lax.axis_index("s") * 2 + lax.axis_index("c")
        base = wid * _ROWS_W
        pltpu.sync_copy(p_hbm.at[pl.ds(base, _ROWS_W)], idx_v)
        pltpu.sync_copy(xf_hbm.at[pl.ds(base, _ROWS_W)], rows_v)
        pltpu.sync_copy(rows_v, out_hbm.at[idx_v])

    return disp(xf, p)


def _combine_call(y_pad, p):
    mesh = plsc.VectorSubcoreMesh(core_axis_name="c", subcore_axis_name="s")

    @functools.partial(
        pl.kernel, mesh=mesh,
        out_type=jax.ShapeDtypeStruct((S, D_MODEL), jnp.float32),
        scratch_types=[
            pltpu.VMEM((_ROWS_W,), jnp.int32),
            pltpu.VMEM((_ROWS_W, D_MODEL), jnp.float32),
            pltpu.SemaphoreType.DMA,
        ],
    )
    def comb(ypad_hbm, p_hbm, out_hbm, idx_v, rows_v, sem):
        wid = lax.axis_index("s") * 2 + lax.axis_index("c")
        base = wid * _ROWS_W
        pltpu.sync_copy(p_hbm.at[pl.ds(base, _ROWS_W)], idx_v)
        pltpu.async_copy(ypad_hbm.at[idx_v], rows_v, sem).wait()
        pltpu.sync_copy(rows_v, out_hbm.at[pl.ds(base, _ROWS_W)])

    return comb(y_pad, p)


def _ffn_body(be_ref, nact_ref, x_ref, w1_ref, b1_ref, w2_ref, b2_ref,
              out_ref):
    b = pl.program_id(0)

    @pl.when(b < nact_ref[0])
    def _active():
        e = be_ref[jnp.minimum(b, nact_ref[0] - 1)]
        x = x_ref[...]
        h = (jnp.dot(x, w1_ref[0], preferred_element_type=jnp.float32,
                     precision=lax.Precision.DEFAULT)
             + b1_ref[pl.ds(e, 1), :])
        h = 0.5 * h * (1.0 + lax.erf(h * 0.7071067811865476))
        out_ref[...] = (jnp.dot(h, w2_ref[0],
                                preferred_element_type=jnp.float32,
                                precision=lax.Precision.DEFAULT)
                        + b2_ref[pl.ds(e, 1), :])


def _ffn_call(block_expert, nact, x_pad, W1, b1, W2, b2):
    def _rowblk(b, be, na):
        return (jnp.minimum(b, na[0] - 1), 0)

    def _expert3(b, be, na):
        return (be[jnp.minimum(b, na[0] - 1)], 0, 0)

    grid_spec = pltpu.PrefetchScalarGridSpec(
        num_scalar_prefetch=2,
        grid=(MAXB,),
        in_specs=[
            pl.BlockSpec((T, D_MODEL), _rowblk),
            pl.BlockSpec((1, D_MODEL, D_FF), _expert3),
            pl.BlockSpec((E, D_FF), lambda b, be, na: (0, 0)),
            pl.BlockSpec((1, D_FF, D_MODEL), _expert3),
            pl.BlockSpec((E, D_MODEL), lambda b, be, na: (0, 0)),
        ],
        out_specs=pl.BlockSpec((T, D_MODEL), _rowblk),
    )
    return pl.pallas_call(
        _ffn_body,
        grid_spec=grid_spec,
        out_shape=jax.ShapeDtypeStruct((P, D_MODEL), jnp.float32),
    )(block_expert, nact, x_pad, W1, b1, W2, b2)


def kernel(x, Wg1, bg1, Wg2, bg2, W1, b1, W2, b2):
    B, S_, D = x.shape
    xf = x.reshape(S_, D)

    p3, be, nact, lb = _route_call(
        xf, Wg1, bg1.reshape(1, -1), Wg2, bg2.reshape(1, -1))
    p = p3.reshape(S_)
    block_expert = be.reshape(MAXB)

    x_pad = _dispatch_call(xf, p)
    y_pad = _ffn_call(block_expert, nact.reshape(1), x_pad, W1, b1, W2, b2)
    y = _combine_call(y_pad, p)

    return y.reshape(B, S_, D), lb.reshape(())
